# baseline (speedup 1.0000x reference)
import functools
import math

import jax
import jax.numpy as jnp
from jax import lax
from jax.experimental import pallas as pl
from jax.experimental.pallas import tpu as pltpu
from jax.experimental.pallas import tpu_sc as plsc

_NW = 32
_G = 128


@jax.jit
def _embed(idx, table_p):
    nw, n_streams_, g_ = idx.shape
    n = nw * n_streams_ * g_
    v, dp = table_p.shape
    d = 64
    scale = math.sqrt(d)
    n_per_w = n // _NW
    n_streams = n_per_w // _G
    mesh = plsc.VectorSubcoreMesh(
        core_axis_name="c", subcore_axis_name="s", num_cores=2, num_subcores=16
    )

    @functools.partial(
        pl.kernel,
        out_type=jax.ShapeDtypeStruct((n, dp), jnp.float32),
        mesh=mesh,
        scratch_types=[
            pltpu.VMEM((n_streams, _G), jnp.int32),
            pltpu.VMEM((_G, dp), jnp.float32),
            pltpu.SemaphoreType.DMA,
        ],
    )
    def emb(idx_hbm, tab_hbm, out_hbm, idx_v, rows_v, sem):
        wid = lax.axis_index("s") * 2 + lax.axis_index("c")
        base = wid * n_per_w
        pltpu.sync_copy(idx_hbm.at[wid], idx_v.at[...])

        def stream(g, carry):
            pltpu.async_copy(tab_hbm.at[idx_v.at[g]], rows_v, sem).wait()

            def srow(r, carry2):
                for j in range(d // 16):
                    sl = pl.ds(j * 16, 16)
                    rows_v[r, sl] = rows_v[r, sl] * scale
                return carry2

            lax.fori_loop(0, _G, srow, 0)
            pltpu.sync_copy(rows_v, out_hbm.at[pl.ds(base + g * _G, _G)])
            return carry

        lax.fori_loop(0, n_streams, stream, 0)

    return emb(idx, table_p)


def kernel(tokens, embedding_weight):
    b, s = tokens.shape
    _, d = embedding_weight.shape
    n = b * s
    idx = tokens.reshape(_NW, n // (_NW * _G), _G).astype(jnp.int32)
    table_p = jnp.pad(embedding_weight, ((0, 0), (0, 128 - d)))
    out = _embed(idx, table_p)
    return out[:, :d].reshape(b, s, d)
